# Initial kernel scaffold; baseline (speedup 1.0000x reference)
import jax
import jax.numpy as jnp
from jax.experimental import pallas as pl


def kernel(shape_ids, color_ids, sym_feats, shape_table, color_table, ln_w, ln_b, fc_w, fc_b):
    raise NotImplementedError("write your pallas kernel here")


# speedup vs baseline: 3.0122x; 3.0122x over previous
import functools

import jax
import jax.numpy as jnp
from jax import lax
from jax.experimental import pallas as pl
from jax.experimental.pallas import tpu as pltpu
from jax.experimental.pallas import tpu_sc as plsc

B, L, D = 4096, 50, 64
NL = 1000

NC, NS, LANES = 2, 16, 16
NW = NC * NS
ROWS_PER_LINE = 2
IDS_PER_LINE = ROWS_PER_LINE * L
LINE_W = 112
N_LINES = B // ROWS_PER_LINE
LINES_PER_W = N_LINES // NW


def _sc_pool_kernel(sid_hbm, cid_hbm, stab_hbm, ctab_hbm, out_hbm,
                    sid_v, cid_v, rows_s, rows_c, acc_v, sems):
    wid = lax.axis_index("s") * NC + lax.axis_index("c")
    line0 = wid * LINES_PER_W

    pltpu.sync_copy(sid_hbm.at[pl.ds(line0, LINES_PER_W)], sid_v)
    pltpu.sync_copy(cid_hbm.at[pl.ds(line0, LINES_PER_W)], cid_v)

    def mask_line(l, carry):
        for j in range(LINE_W // LANES):
            s = sid_v[l, pl.ds(j * LANES, LANES)]
            c = cid_v[l, pl.ds(j * LANES, LANES)]
            cid_v[l, pl.ds(j * LANES, LANES)] = jnp.where(s == 0, 0, c)
        return carry
    lax.fori_loop(0, LINES_PER_W, mask_line, 0)

    def line_body(k, carry):
        ds = pltpu.async_copy(stab_hbm.at[sid_v.at[k]], rows_s, sems.at[0])
        dc = pltpu.async_copy(ctab_hbm.at[cid_v.at[k]], rows_c, sems.at[1])
        ds.wait()
        dc.wait()
        for r in range(ROWS_PER_LINE):
            acc = [jnp.zeros((LANES,), jnp.float32) for _ in range(D // LANES)]
            for j in range(L):
                row = r * L + j
                for q in range(D // LANES):
                    acc[q] = acc[q] + rows_s[row, pl.ds(q * LANES, LANES)] \
                                    + rows_c[row, pl.ds(q * LANES, LANES)]
            out_row = ROWS_PER_LINE * k + r
            for q in range(D // LANES):
                acc_v[out_row, pl.ds(q * LANES, LANES)] = acc[q]
        return carry
    lax.fori_loop(0, LINES_PER_W, line_body, 0)

    pltpu.sync_copy(acc_v, out_hbm.at[pl.ds(line0 * ROWS_PER_LINE,
                                            LINES_PER_W * ROWS_PER_LINE)])


def _sc_pool(sid_p, cid_p, shape_table, color_table):
    mesh = plsc.VectorSubcoreMesh(core_axis_name="c", subcore_axis_name="s",
                                  num_cores=NC, num_subcores=NS)
    return pl.kernel(
        _sc_pool_kernel,
        out_type=jax.ShapeDtypeStruct((B, D), jnp.float32),
        mesh=mesh,
        compiler_params=pltpu.CompilerParams(use_tc_tiling_on_sc=False),
        scratch_types=[
            pltpu.VMEM((LINES_PER_W, LINE_W), jnp.int32),
            pltpu.VMEM((LINES_PER_W, LINE_W), jnp.int32),
            pltpu.VMEM((LINE_W, D), jnp.float32),
            pltpu.VMEM((LINE_W, D), jnp.float32),
            pltpu.VMEM((LINES_PER_W * ROWS_PER_LINE, D), jnp.float32),
            pltpu.SemaphoreType.DMA((2,)),
        ],
    )(sid_p, cid_p, shape_table, color_table)


BB = 512


def _tc_head_kernel(pooled_ref, sid_ref, sym_ref, lnw_ref, lnb_ref,
                    w1_ref, w2_ref, b_ref, out_ref):
    sid = sid_ref[...]
    cnt = jnp.sum((sid != 0).astype(jnp.float32), axis=1, keepdims=True)
    denom = jnp.maximum(cnt, 1.0)
    pooled = pooled_ref[...] / denom

    sym = sym_ref[...]
    lane = lax.broadcasted_iota(jnp.int32, sym.shape, 1)
    valid = lane < 3
    mu = jnp.sum(sym, axis=1, keepdims=True) * (1.0 / 3.0)
    d0 = jnp.where(valid, sym - mu, 0.0)
    var = jnp.sum(d0 * d0, axis=1, keepdims=True) * (1.0 / 3.0)
    syn = jnp.where(valid,
                    d0 * lax.rsqrt(var + 1e-5) * lnw_ref[...] + lnb_ref[...],
                    0.0)

    out_ref[...] = (
        lax.dot_general(pooled, w1_ref[...], (((1,), (1,)), ((), ())),
                        preferred_element_type=jnp.float32)
        + lax.dot_general(syn, w2_ref[...], (((1,), (1,)), ((), ())),
                          preferred_element_type=jnp.float32)
        + b_ref[...]
    )


def _tc_head(pooled, shape_ids, sym8, lnw8, lnb8, w1, w2, b2):
    grid = (B // BB,)
    return pl.pallas_call(
        _tc_head_kernel,
        grid=grid,
        in_specs=[
            pl.BlockSpec((BB, D), lambda i: (i, 0)),
            pl.BlockSpec((BB, L), lambda i: (i, 0)),
            pl.BlockSpec((BB, 8), lambda i: (i, 0)),
            pl.BlockSpec((1, 8), lambda i: (0, 0)),
            pl.BlockSpec((1, 8), lambda i: (0, 0)),
            pl.BlockSpec((NL, D), lambda i: (0, 0)),
            pl.BlockSpec((NL, 8), lambda i: (0, 0)),
            pl.BlockSpec((1, NL), lambda i: (0, 0)),
        ],
        out_specs=pl.BlockSpec((BB, NL), lambda i: (i, 0)),
        out_shape=jax.ShapeDtypeStruct((B, NL), jnp.float32),
    )(pooled, shape_ids, sym8, lnw8, lnb8, w1, w2, b2)


def kernel(shape_ids, color_ids, sym_feats, shape_table, color_table,
           ln_w, ln_b, fc_w, fc_b):
    sid_p = jnp.pad(shape_ids.reshape(N_LINES, IDS_PER_LINE),
                    ((0, 0), (0, LINE_W - IDS_PER_LINE)))
    cid_p = jnp.pad(color_ids.reshape(N_LINES, IDS_PER_LINE),
                    ((0, 0), (0, LINE_W - IDS_PER_LINE)))
    pooled_sum = _sc_pool(sid_p, cid_p, shape_table, color_table)

    sym8 = jnp.pad(sym_feats, ((0, 0), (0, 5)))
    lnw8 = jnp.pad(ln_w, (0, 5)).reshape(1, 8)
    lnb8 = jnp.pad(ln_b, (0, 5)).reshape(1, 8)
    w1 = fc_w[:, :D]
    w2 = jnp.pad(fc_w[:, D:], ((0, 0), (0, 5)))
    b2 = fc_b.reshape(1, NL)
    return _tc_head(pooled_sum, shape_ids, sym8, lnw8, lnb8, w1, w2, b2)
